# baseline (speedup 1.0000x reference)
import functools

import jax
import jax.numpy as jnp
from jax.experimental import pallas as pl
from jax.experimental.pallas import tpu as pltpu


def _bf(v):
    return v.astype(jnp.bfloat16).astype(jnp.float32)


def _stein_body(xt_ref, st_ref, stb_ref, out_ref, obj_ref, stbf_ref,
                *, msel, dim, n):
    fdim = jnp.float32(dim)
    nblk = n // 128
    lane_iota = jax.lax.broadcasted_iota(jnp.int32, (1, n), 1)
    lane128 = jax.lax.broadcasted_iota(jnp.int32, (1, 128), 1)
    out_iota = jax.lax.broadcasted_iota(jnp.int32, (1, msel), 1)

    stbf_ref[...] = stb_ref[...].astype(jnp.float32)

    def argmin_first(o):
        mv = jnp.min(o)
        return jnp.min(jnp.where(o == mv, lane_iota, jnp.int32(n)))

    stv = st_ref[...]
    obj_ref[...] = fdim + jnp.sum(stv * stv, axis=0, keepdims=True)
    p0 = argmin_first(obj_ref[...])
    idx0 = jnp.where(out_iota == 0, p0, jnp.zeros((1, msel), jnp.int32))

    col_iota = jax.lax.broadcasted_iota(jnp.int32, (dim, 128), 1)

    def select_col(ref, base, pm):
        blk = ref[:, pl.ds(base, 128)]
        return jnp.sum(jnp.where(col_iota == pm, blk, 0.0), axis=1,
                       keepdims=True)

    def step(t, carry):
        p, idx_acc = carry
        base = pl.multiple_of((p // 128) * 128, 128)
        pm = p % 128
        xi = select_col(xt_ref, base, pm)
        si = select_col(st_ref, base, pm)
        sib = _bf(si)

        def g_body(g, acc):
            vmn, vmi = acc
            c0 = pl.multiple_of(g * 128, 128)
            z = jnp.zeros((8, 128), jnp.float32)
            r2a, sida, sjda, ssa = z, z, z, z
            for k in range(dim // 8):
                sl = slice(k * 8, (k + 1) * 8)
                xib = jnp.broadcast_to(xi[sl, :], (8, 128))
                sibb = jnp.broadcast_to(sib[sl, :], (8, 128))
                xt_k = xt_ref[sl, pl.ds(c0, 128)]
                st_k = st_ref[sl, pl.ds(c0, 128)]
                sb_k = stbf_ref[sl, pl.ds(c0, 128)]
                xd = xib - xt_k
                xdb = _bf(xd)
                r2a = r2a + xd * xd
                sida = sida + sibb * xdb
                sjda = sjda + st_k * xd
                ssa = ssa + sibb * sb_k
            r2 = jnp.sum(r2a, axis=0, keepdims=True)
            sid = jnp.sum(sida, axis=0, keepdims=True)
            sjd = jnp.sum(sjda, axis=0, keepdims=True)
            ss = jnp.sum(ssa, axis=0, keepdims=True)
            q = 1.0 + r2
            qs = jax.lax.rsqrt(q)
            q15 = qs * qs * qs
            q25 = q15 * qs * qs
            ki = (fdim * q15 - 3.0 * r2 * q25) + (sid - sjd) * q15 + ss * qs
            ob = obj_ref[0:1, pl.ds(c0, 128)] + 2.0 * ki
            obj_ref[0:1, pl.ds(c0, 128)] = ob
            upd = ob < vmn
            vmn = jnp.where(upd, ob, vmn)
            vmi = jnp.where(upd, lane128 + c0, vmi)
            return (vmn, vmi)

        vmn0 = jnp.full((1, 128), jnp.inf, jnp.float32)
        vmi0 = jnp.full((1, 128), n, jnp.int32)
        vmn, vmi = jax.lax.fori_loop(0, nblk, g_body, (vmn0, vmi0))
        mv = jnp.min(vmn)
        pn = jnp.min(jnp.where(vmn == mv, vmi, jnp.int32(n)))
        idx_acc = jnp.where(out_iota == t, pn, idx_acc)
        return (pn, idx_acc)

    _, idx_acc = jax.lax.fori_loop(1, msel, step, (p0, idx0))
    out_ref[...] = idx_acc


def kernel(x, score_p, m):
    n, dim = x.shape
    msel = int(max(1, min(256, n)))
    xt = x.T
    st = score_p.T
    stb = st.astype(jnp.bfloat16)
    out = pl.pallas_call(
        functools.partial(_stein_body, msel=msel, dim=dim, n=n),
        out_shape=jax.ShapeDtypeStruct((1, msel), jnp.int32),
        scratch_shapes=[pltpu.VMEM((1, n), jnp.float32),
                        pltpu.VMEM((dim, n), jnp.float32)],
    )(xt, st, stb)
    return out.reshape(msel)


# speedup vs baseline: 1.5895x; 1.5895x over previous
import functools

import jax
import jax.numpy as jnp
from jax.experimental import pallas as pl
from jax.experimental.pallas import tpu as pltpu


def _bf(v):
    return v.astype(jnp.bfloat16).astype(jnp.float32)


def _stein_body(xt_ref, st_ref, stb_ref, out_ref, obj_ref, stbf_ref,
                *, msel, dim, n):
    fdim = jnp.float32(dim)
    BLK = 512
    lane_iota = jax.lax.broadcasted_iota(jnp.int32, (1, n), 1)
    laneB = jax.lax.broadcasted_iota(jnp.int32, (1, BLK), 1)
    out_iota = jax.lax.broadcasted_iota(jnp.int32, (1, msel), 1)

    stbf_ref[...] = stb_ref[...].astype(jnp.float32)

    def argmin_first(o):
        mv = jnp.min(o)
        return jnp.min(jnp.where(o == mv, lane_iota, jnp.int32(n)))

    stv = st_ref[...]
    obj_ref[...] = fdim + jnp.sum(stv * stv, axis=0, keepdims=True)
    p0 = argmin_first(obj_ref[...])
    idx0 = jnp.where(out_iota == 0, p0, jnp.zeros((1, msel), jnp.int32))

    col_iota = jax.lax.broadcasted_iota(jnp.int32, (dim, 128), 1)

    def select_col(ref, base, pm):
        blk = ref[:, pl.ds(base, 128)]
        return jnp.sum(jnp.where(col_iota == pm, blk, 0.0), axis=1,
                       keepdims=True)

    def step(t, carry):
        p, idx_acc = carry
        base = pl.multiple_of((p // 128) * 128, 128)
        pm = p % 128
        xi = select_col(xt_ref, base, pm)
        si = select_col(st_ref, base, pm)
        sib = _bf(si)

        def g_body(g, acc):
            vmn, vmi = acc
            c0 = pl.multiple_of(g * BLK, BLK)
            z = jnp.zeros((8, BLK), jnp.float32)
            r2a = [z, z]
            sida = [z, z]
            sjda = [z, z]
            ssa = [z, z]
            for k in range(dim // 8):
                u = k & 1
                sl = slice(k * 8, (k + 1) * 8)
                xib = jnp.broadcast_to(xi[sl, :], (8, BLK))
                sibb = jnp.broadcast_to(sib[sl, :], (8, BLK))
                xt_k = xt_ref[sl, pl.ds(c0, BLK)]
                st_k = st_ref[sl, pl.ds(c0, BLK)]
                sb_k = stbf_ref[sl, pl.ds(c0, BLK)]
                xd = xib - xt_k
                xdb = _bf(xd)
                r2a[u] = r2a[u] + xd * xd
                sida[u] = sida[u] + sibb * xdb
                sjda[u] = sjda[u] + st_k * xd
                ssa[u] = ssa[u] + sibb * sb_k
            r2 = jnp.sum(r2a[0] + r2a[1], axis=0, keepdims=True)
            sid = jnp.sum(sida[0] + sida[1], axis=0, keepdims=True)
            sjd = jnp.sum(sjda[0] + sjda[1], axis=0, keepdims=True)
            ss = jnp.sum(ssa[0] + ssa[1], axis=0, keepdims=True)
            q = 1.0 + r2
            qs = jax.lax.rsqrt(q)
            q15 = qs * qs * qs
            q25 = q15 * qs * qs
            ki = (fdim * q15 - 3.0 * r2 * q25) + (sid - sjd) * q15 + ss * qs
            ob = obj_ref[0:1, pl.ds(c0, BLK)] + 2.0 * ki
            obj_ref[0:1, pl.ds(c0, BLK)] = ob
            upd = ob < vmn
            vmn = jnp.where(upd, ob, vmn)
            vmi = jnp.where(upd, laneB + c0, vmi)
            return (vmn, vmi)

        vmn0 = jnp.full((1, BLK), jnp.inf, jnp.float32)
        vmi0 = jnp.full((1, BLK), n, jnp.int32)
        vmn, vmi = jax.lax.fori_loop(0, n // BLK, g_body, (vmn0, vmi0))
        mv = jnp.min(vmn)
        pn = jnp.min(jnp.where(vmn == mv, vmi, jnp.int32(n)))
        idx_acc = jnp.where(out_iota == t, pn, idx_acc)
        return (pn, idx_acc)

    _, idx_acc = jax.lax.fori_loop(1, msel, step, (p0, idx0))
    out_ref[...] = idx_acc


def kernel(x, score_p, m):
    n, dim = x.shape
    msel = int(max(1, min(256, n)))
    xt = x.T
    st = score_p.T
    stb = st.astype(jnp.bfloat16)
    out = pl.pallas_call(
        functools.partial(_stein_body, msel=msel, dim=dim, n=n),
        out_shape=jax.ShapeDtypeStruct((1, msel), jnp.int32),
        scratch_shapes=[pltpu.VMEM((1, n), jnp.float32),
                        pltpu.VMEM((dim, n), jnp.float32)],
    )(xt, st, stb)
    return out.reshape(msel)
